# Initial kernel scaffold; baseline (speedup 1.0000x reference)
import jax
import jax.numpy as jnp
from jax.experimental import pallas as pl


def kernel(x, w1, w2):
    raise NotImplementedError("write your pallas kernel here")


# speedup vs baseline: 1.0068x; 1.0068x over previous
import functools

import jax
import jax.numpy as jnp
from jax.experimental import pallas as pl
from jax.experimental.pallas import tpu as pltpu


def _ca_fused_kernel(x_ref, w1t_ref, w2t_ref, o_ref, *, bb, c, hw):
    x = x_ref[...]

    xf = x.reshape(bb * c, hw).astype(jnp.float32)
    ones_col = jnp.ones((hw, 1), jnp.float32)
    s = jnp.dot(xf, ones_col, preferred_element_type=jnp.float32)
    avg = s.reshape(bb, c) * (1.0 / hw)

    mx = jnp.max(x, axis=-1).astype(jnp.float32)

    pooled = jnp.concatenate([avg, mx], axis=0)
    h = jnp.maximum(
        jnp.dot(pooled, w1t_ref[...], preferred_element_type=jnp.float32), 0.0)
    y = jnp.dot(h, w2t_ref[...], preferred_element_type=jnp.float32)
    gate = jax.nn.sigmoid(y[:bb] + y[bb:])

    o_ref[...] = (x * gate.astype(x.dtype)[:, :, None]).astype(o_ref.dtype)


@jax.jit
def _channel_attention(x, w1, w2):
    B, C, H, W = x.shape
    Cr = w1.shape[0]
    HW = H * W
    x_flat = x.reshape(B, C, HW)

    w1t = jnp.asarray(w1, jnp.float32).T
    w2t = jnp.asarray(w2, jnp.float32).T

    itemsize = jnp.dtype(x.dtype).itemsize
    bytes_per_batch = C * HW * itemsize
    bb = max(1, min(B, (8 << 20) // max(1, bytes_per_batch)))
    bb = 1 << (bb.bit_length() - 1)
    while bb > 1 and 4 * bb * bytes_per_batch > (44 << 20):
        bb //= 2
    if B >= 4:
        bb = min(bb, B // 4)
    grid_b = pl.cdiv(B, bb)

    vmem_limit = int(min(4 * bb * bytes_per_batch + 8 * (w1t.size + w2t.size)
                         + (6 << 20), 56 << 20))

    out_flat = pl.pallas_call(
        functools.partial(_ca_fused_kernel, bb=bb, c=C, hw=HW),
        out_shape=jax.ShapeDtypeStruct((B, C, HW), x.dtype),
        grid=(grid_b,),
        in_specs=[
            pl.BlockSpec((bb, C, HW), lambda b: (b, 0, 0)),
            pl.BlockSpec((C, Cr), lambda b: (0, 0)),
            pl.BlockSpec((Cr, C), lambda b: (0, 0)),
        ],
        out_specs=pl.BlockSpec((bb, C, HW), lambda b: (b, 0, 0)),
        compiler_params=pltpu.CompilerParams(
            dimension_semantics=("parallel",),
            vmem_limit_bytes=vmem_limit),
    )(x_flat, w1t, w2t)
    return out_flat.reshape(B, C, H, W)


def kernel(x, w1, w2):
    return _channel_attention(x, w1, w2)
